# Initial kernel scaffold; baseline (speedup 1.0000x reference)
import jax
import jax.numpy as jnp
from jax.experimental import pallas as pl


def kernel(x, table, W, b):
    raise NotImplementedError("write your pallas kernel here")


# speedup vs baseline: 12.3462x; 12.3462x over previous
import functools

import jax
import jax.numpy as jnp
from jax import lax
from jax.experimental import pallas as pl
from jax.experimental.pallas import tpu as pltpu
from jax.experimental.pallas import tpu_sc as plsc

_VOCAB = 1_000_000
_EMB = 32
_B = 16384
_L = 200

_NC = 2
_NS = 16
_LN = 16
_NW = _NC * _NS
_RW = _B // _NW
_RC = 256
_NCH = _RW // _RC
_C = _RC * _L
_CR = _C // 128

_VR = _VOCAB // 8
_VB = 5000


def _tw_body(t_ref, w_ref, b_ref, o_ref):
    y = t_ref[...] * w_ref[...]
    o_ref[...] = y.reshape(_VB, 8, _EMB).sum(axis=-1) + b_ref[0, 0]


_tw_call = pl.pallas_call(
    _tw_body,
    grid=(_VR // _VB,),
    in_specs=[
        pl.BlockSpec((_VB, 8 * _EMB), lambda i: (i, 0)),
        pl.BlockSpec((1, 8 * _EMB), lambda i: (0, 0)),
        pl.BlockSpec((1, 1), lambda i: (0, 0)),
    ],
    out_specs=pl.BlockSpec((_VB, 8), lambda i: (i, 0)),
    out_shape=jax.ShapeDtypeStruct((_VR, 8), jnp.float32),
)


def _permute(a, perm):
    dn = lax.GatherDimensionNumbers(
        offset_dims=(), collapsed_slice_dims=(0,), start_index_map=(0,)
    )
    return lax.gather(
        a, perm[:, None], dn, (1,),
        mode=lax.GatherScatterMode.PROMISE_IN_BOUNDS,
    )


@functools.partial(
    pl.kernel,
    out_type=jax.ShapeDtypeStruct((_B,), jnp.float32),
    mesh=plsc.VectorSubcoreMesh(core_axis_name="c", subcore_axis_name="s"),
    scratch_types=[
        pltpu.VMEM((_C + _LN,), jnp.int32),
        pltpu.VMEM((_C + _LN,), jnp.float32),
        pltpu.VMEM((_RC,), jnp.float32),
        pltpu.SemaphoreType.DMA,
    ],
)
def _pool(x_hbm, tw_hbm, out_hbm, idx_v, vals_v, out_v, sem):
    wid = lax.axis_index("s") * _NC + lax.axis_index("c")
    lane = lax.iota(jnp.int32, _LN)
    lanemask = lane < 8
    perms = [lane ^ (1 << p) for p in range(4)]
    idx_v[pl.ds(_C, _LN)] = jnp.zeros((_LN,), jnp.int32)

    for ch in range(_NCH):
        rowbase = wid * _RW + ch * _RC
        pltpu.sync_copy(x_hbm.at[pl.ds(rowbase * _L, _C)], idx_v.at[pl.ds(0, _C)])
        pltpu.async_copy(tw_hbm.at[idx_v], vals_v, sem).wait()

        def group(g, carry):
            sums = jnp.zeros((_LN,), jnp.float32)
            for i in range(_LN):
                base = (g * _LN + i) * _L
                acc = vals_v[pl.ds(base, _LN)]
                for k in range(1, 12):
                    acc = acc + vals_v[pl.ds(base + k * _LN, _LN)]
                tail = vals_v[pl.ds(base + 12 * _LN, _LN)]
                acc = acc + jnp.where(lanemask, tail, 0.0)
                for p in perms:
                    acc = acc + _permute(acc, p)
                sums = jnp.where(lane == i, acc, sums)
            z = sums * (1.0 / _L)
            out_v[pl.ds(g * _LN, _LN)] = 1.0 / (1.0 + jnp.exp(-z))
            return carry

        lax.fori_loop(0, _RC // _LN, group, 0)
        pltpu.sync_copy(out_v, out_hbm.at[pl.ds(rowbase, _RC)])


def kernel(x, table, W, b):
    w256 = jnp.tile(W.reshape(1, _EMB), (1, 8))
    tw = _tw_call(table.reshape(_VR, 8 * _EMB), w256, b.reshape(1, 1))
    out = _pool(x.reshape(_B * _L), tw.reshape(_VOCAB))
    return out.reshape(_B, 1)


# speedup vs baseline: 45.8035x; 3.7099x over previous
import functools

import jax
import jax.numpy as jnp
from jax import lax
from jax.experimental import pallas as pl
from jax.experimental.pallas import tpu as pltpu
from jax.experimental.pallas import tpu_sc as plsc

_VOCAB = 1_000_000
_EMB = 32
_B = 16384
_L = 200

_NC = 2
_NS = 16
_LN = 16
_NW = _NC * _NS
_RW = _B // _NW
_RC = 256
_NCH = _RW // _RC
_C = _RC * _L
_CR = _C // 128

_VP = 1_024_000
_VB = 16384
_NFULL = _VOCAB // _VB
_VMAIN = _NFULL * _VB
_VTAIL = _VOCAB - _VMAIN


def _tw_body(t_ref, w_ref, b_ref, o_ref):
    o_ref[...] = jnp.sum(t_ref[...] * w_ref[...], axis=0) + b_ref[0, 0]


_tw_main = pl.pallas_call(
    _tw_body,
    grid=(_NFULL,),
    in_specs=[
        pl.BlockSpec((_EMB, _VB), lambda i: (0, i)),
        pl.BlockSpec((_EMB, 1), lambda i: (0, 0)),
        pl.BlockSpec((1, 1), lambda i: (0, 0)),
    ],
    out_specs=pl.BlockSpec((_VB,), lambda i: (i,)),
    out_shape=jax.ShapeDtypeStruct((_VP,), jnp.float32),
)

_tw_tail = pl.pallas_call(
    _tw_body,
    grid=(1,),
    in_specs=[
        pl.BlockSpec((_EMB, _VTAIL), lambda i: (0, 0)),
        pl.BlockSpec((_EMB, 1), lambda i: (0, 0)),
        pl.BlockSpec((1, 1), lambda i: (0, 0)),
    ],
    out_specs=pl.BlockSpec((_VTAIL,), lambda i: (0,)),
    out_shape=jax.ShapeDtypeStruct((_VTAIL,), jnp.float32),
)


def _permute(a, perm):
    dn = lax.GatherDimensionNumbers(
        offset_dims=(), collapsed_slice_dims=(0,), start_index_map=(0,)
    )
    return lax.gather(
        a, perm[:, None], dn, (1,),
        mode=lax.GatherScatterMode.PROMISE_IN_BOUNDS,
    )


@functools.partial(
    pl.kernel,
    out_type=jax.ShapeDtypeStruct((_B,), jnp.float32),
    mesh=plsc.VectorSubcoreMesh(core_axis_name="c", subcore_axis_name="s"),
    scratch_types=[
        pltpu.VMEM((_C + _LN,), jnp.int32),
        pltpu.VMEM((_C + _LN,), jnp.float32),
        pltpu.VMEM((_RC,), jnp.float32),
        pltpu.SemaphoreType.DMA,
    ],
)
def _pool(x_hbm, tw_hbm, out_hbm, idx_v, vals_v, out_v, sem):
    wid = lax.axis_index("s") * _NC + lax.axis_index("c")
    lane = lax.iota(jnp.int32, _LN)
    lanemask = lane < 8
    perms = [lane ^ (1 << p) for p in range(4)]
    idx_v[pl.ds(_C, _LN)] = jnp.zeros((_LN,), jnp.int32)

    for ch in range(_NCH):
        rowbase = wid * _RW + ch * _RC
        pltpu.sync_copy(x_hbm.at[pl.ds(rowbase * _L, _C)], idx_v.at[pl.ds(0, _C)])
        pltpu.async_copy(tw_hbm.at[idx_v], vals_v, sem).wait()

        def group(g, carry):
            sums = jnp.zeros((_LN,), jnp.float32)
            for i in range(_LN):
                base = (g * _LN + i) * _L
                acc = vals_v[pl.ds(base, _LN)]
                for k in range(1, 12):
                    acc = acc + vals_v[pl.ds(base + k * _LN, _LN)]
                tail = vals_v[pl.ds(base + 12 * _LN, _LN)]
                acc = acc + jnp.where(lanemask, tail, 0.0)
                for p in perms:
                    acc = acc + _permute(acc, p)
                sums = jnp.where(lane == i, acc, sums)
            z = sums * (1.0 / _L)
            out_v[pl.ds(g * _LN, _LN)] = 1.0 / (1.0 + jnp.exp(-z))
            return carry

        lax.fori_loop(0, _RC // _LN, group, 0)
        pltpu.sync_copy(out_v, out_hbm.at[pl.ds(rowbase, _RC)])


def kernel(x, table, W, b):
    tt = table.T
    wc = W.reshape(_EMB, 1)
    bb = b.reshape(1, 1)
    tw = _tw_main(tt, wc, bb)
    tail = _tw_tail(lax.slice(tt, (0, _VMAIN), (_EMB, _VOCAB)), wc, bb)
    tw = lax.dynamic_update_slice(tw, tail, (_VMAIN,))
    out = _pool(x.reshape(_B * _L), tw)
    return out.reshape(_B, 1)


# speedup vs baseline: 60.3345x; 1.3172x over previous
import functools

import jax
import jax.numpy as jnp
from jax import lax
from jax.experimental import pallas as pl
from jax.experimental.pallas import tpu as pltpu
from jax.experimental.pallas import tpu_sc as plsc

_VOCAB = 1_000_000
_EMB = 32
_B = 16384
_L = 200

_NC = 2
_NS = 16
_LN = 16
_NW = _NC * _NS
_RW = _B // _NW
_RC = 128
_NCH = _RW // _RC
_NG = _RC // _LN

_VP = 1_024_000
_VB = 65536
_NFULL = _VOCAB // _VB
_VMAIN = _NFULL * _VB
_VTAIL = _VOCAB - _VMAIN


def _tw_body(t_ref, w_ref, b_ref, o_ref):
    o_ref[...] = jnp.sum(t_ref[...] * w_ref[...], axis=0) + b_ref[0, 0]


_tw_main = pl.pallas_call(
    _tw_body,
    grid=(_NFULL,),
    in_specs=[
        pl.BlockSpec((_EMB, _VB), lambda i: (0, i)),
        pl.BlockSpec((_EMB, 1), lambda i: (0, 0)),
        pl.BlockSpec((1, 1), lambda i: (0, 0)),
    ],
    out_specs=pl.BlockSpec((_VB,), lambda i: (i,)),
    out_shape=jax.ShapeDtypeStruct((_VP,), jnp.float32),
)

_tw_tail = pl.pallas_call(
    _tw_body,
    grid=(1,),
    in_specs=[
        pl.BlockSpec((_EMB, _VTAIL), lambda i: (0, 0)),
        pl.BlockSpec((_EMB, 1), lambda i: (0, 0)),
        pl.BlockSpec((1, 1), lambda i: (0, 0)),
    ],
    out_specs=pl.BlockSpec((_VTAIL,), lambda i: (0,)),
    out_shape=jax.ShapeDtypeStruct((_VTAIL,), jnp.float32),
)


@functools.partial(
    pl.kernel,
    out_type=jax.ShapeDtypeStruct((_B,), jnp.float32),
    mesh=plsc.VectorSubcoreMesh(core_axis_name="c", subcore_axis_name="s"),
    scratch_types=[
        pltpu.VMEM((2, _L, _RC), jnp.int32),
        pltpu.VMEM((2, _L, _RC), jnp.float32),
        pltpu.VMEM((_RW,), jnp.float32),
        pltpu.SemaphoreType.DMA,
        pltpu.SemaphoreType.DMA,
    ],
)
def _pool(xt_hbm, tw_hbm, out_hbm, idx_v, vals_v, out_v, sem0, sem1):
    wid = lax.axis_index("s") * _NC + lax.axis_index("c")
    sems = (sem0, sem1)

    def stage(ch, par):
        rowbase = wid * _RW + ch * _RC
        pltpu.sync_copy(xt_hbm.at[:, pl.ds(rowbase, _RC)], idx_v.at[par])

        def fire(j, carry):
            pltpu.make_async_copy(
                tw_hbm.at[idx_v.at[par, j]], vals_v.at[par, j], sems[par]
            ).start()
            return carry

        lax.fori_loop(0, _L, fire, 0)

    def drain(par):
        def one(j, carry):
            pltpu.make_async_copy(
                tw_hbm.at[idx_v.at[par, j]], vals_v.at[par, j], sems[par]
            ).wait()
            return carry

        lax.fori_loop(0, _L, one, 0)

    def compute(ch, par):
        def body(j, accs):
            return tuple(
                accs[g] + vals_v[par, j, pl.ds(g * _LN, _LN)]
                for g in range(_NG)
            )

        accs = lax.fori_loop(
            0, _L, body,
            tuple(jnp.zeros((_LN,), jnp.float32) for _ in range(_NG)),
            unroll=2,
        )
        for g in range(_NG):
            z = accs[g] * (1.0 / _L)
            out_v[pl.ds(ch * _RC + g * _LN, _LN)] = 1.0 / (1.0 + jnp.exp(-z))

    stage(0, 0)
    for ch in range(_NCH):
        if ch + 1 < _NCH:
            stage(ch + 1, (ch + 1) % 2)
        drain(ch % 2)
        compute(ch, ch % 2)
    pltpu.sync_copy(out_v, out_hbm.at[pl.ds(wid * _RW, _RW)])


def kernel(x, table, W, b):
    tt = table.T
    wc = W.reshape(_EMB, 1)
    bb = b.reshape(1, 1)
    tw = _tw_main(tt, wc, bb)
    tail = _tw_tail(lax.slice(tt, (0, _VMAIN), (_EMB, _VOCAB)), wc, bb)
    tw = lax.dynamic_update_slice(tw, tail, (_VMAIN,))
    out = _pool(x.T, tw)
    return out.reshape(_B, 1)


# speedup vs baseline: 60.5208x; 1.0031x over previous
import functools

import jax
import jax.numpy as jnp
from jax import lax
from jax.experimental import pallas as pl
from jax.experimental.pallas import tpu as pltpu
from jax.experimental.pallas import tpu_sc as plsc

_VOCAB = 1_000_000
_EMB = 32
_B = 16384
_L = 200

_NC = 2
_NS = 16
_LN = 16
_NW = _NC * _NS
_RW = _B // _NW
_RC = 128
_NCH = _RW // _RC
_NG = _RC // _LN

_VP = 1_024_000
_VB = 98304
_NFULL = _VOCAB // _VB
_VMAIN = _NFULL * _VB
_VTAIL = _VOCAB - _VMAIN


def _tw_body(t_ref, w_ref, b_ref, o_ref):
    o_ref[...] = jnp.sum(t_ref[...] * w_ref[...], axis=0) + b_ref[0, 0]


_tw_main = pl.pallas_call(
    _tw_body,
    grid=(_NFULL,),
    in_specs=[
        pl.BlockSpec((_EMB, _VB), lambda i: (0, i)),
        pl.BlockSpec((_EMB, 1), lambda i: (0, 0)),
        pl.BlockSpec((1, 1), lambda i: (0, 0)),
    ],
    out_specs=pl.BlockSpec((_VB,), lambda i: (i,)),
    out_shape=jax.ShapeDtypeStruct((_VP,), jnp.float32),
)

_tw_tail = pl.pallas_call(
    _tw_body,
    grid=(1,),
    in_specs=[
        pl.BlockSpec((_EMB, _VTAIL), lambda i: (0, 0)),
        pl.BlockSpec((_EMB, 1), lambda i: (0, 0)),
        pl.BlockSpec((1, 1), lambda i: (0, 0)),
    ],
    out_specs=pl.BlockSpec((_VTAIL,), lambda i: (0,)),
    out_shape=jax.ShapeDtypeStruct((_VTAIL,), jnp.float32),
)


@functools.partial(
    pl.kernel,
    out_type=jax.ShapeDtypeStruct((_B,), jnp.float32),
    mesh=plsc.VectorSubcoreMesh(core_axis_name="c", subcore_axis_name="s"),
    scratch_types=[
        pltpu.VMEM((2, _L, _RC), jnp.int32),
        pltpu.VMEM((2, _L, _RC), jnp.float32),
        pltpu.VMEM((_RW,), jnp.float32),
        pltpu.SemaphoreType.DMA,
        pltpu.SemaphoreType.DMA,
    ],
)
def _pool(xt_hbm, tw_hbm, out_hbm, idx_v, vals_v, out_v, sem0, sem1):
    wid = lax.axis_index("s") * _NC + lax.axis_index("c")
    sems = (sem0, sem1)

    def stage(ch, par):
        rowbase = wid * _RW + ch * _RC
        pltpu.sync_copy(xt_hbm.at[:, pl.ds(rowbase, _RC)], idx_v.at[par])

        def fire(j, carry):
            pltpu.make_async_copy(
                tw_hbm.at[idx_v.at[par, j]], vals_v.at[par, j], sems[par]
            ).start()
            return carry

        lax.fori_loop(0, _L, fire, 0)

    def drain(par):
        def one(j, carry):
            pltpu.make_async_copy(
                tw_hbm.at[idx_v.at[par, j]], vals_v.at[par, j], sems[par]
            ).wait()
            return carry

        lax.fori_loop(0, _L, one, 0)

    def compute(ch, par):
        def body(j, accs):
            return tuple(
                accs[g] + vals_v[par, j, pl.ds(g * _LN, _LN)]
                for g in range(_NG)
            )

        accs = lax.fori_loop(
            0, _L, body,
            tuple(jnp.zeros((_LN,), jnp.float32) for _ in range(_NG)),
            unroll=2,
        )
        for g in range(_NG):
            z = accs[g] * (1.0 / _L)
            out_v[pl.ds(ch * _RC + g * _LN, _LN)] = 1.0 / (1.0 + jnp.exp(-z))

    stage(0, 0)
    for ch in range(_NCH):
        if ch + 1 < _NCH:
            stage(ch + 1, (ch + 1) % 2)
        drain(ch % 2)
        compute(ch, ch % 2)
    pltpu.sync_copy(out_v, out_hbm.at[pl.ds(wid * _RW, _RW)])


def kernel(x, table, W, b):
    tt = table.T
    wc = W.reshape(_EMB, 1)
    bb = b.reshape(1, 1)
    tw = _tw_main(tt, wc, bb)
    tail = _tw_tail(lax.slice(tt, (0, _VMAIN), (_EMB, _VOCAB)), wc, bb)
    tw = lax.dynamic_update_slice(tw, tail, (_VMAIN,))
    out = _pool(x.T, tw)
    return out.reshape(_B, 1)
